# baseline (speedup 1.0000x reference)
import functools

import jax
import jax.numpy as jnp
from jax.experimental import pallas as pl
from jax.experimental.pallas import tpu as pltpu

TOPK = 8
NEXP = 64
R = 512
NBUF = 2


def _router_block(x_ref, probs_ref, idx_ref, w_ref):
    xb = x_ref[...]
    wb = w_ref[...]
    logits = jax.lax.dot_general(
        wb, xb, (((1,), (1,)), ((), ())), preferred_element_type=jnp.float32
    )

    sub = jax.lax.broadcasted_iota(jnp.int32, logits.shape, 0)
    vals = logits
    top_v = []
    top_i = []
    for _ in range(TOPK):
        mv = jnp.max(vals, axis=0, keepdims=True)
        mi = jnp.min(jnp.where(vals == mv, sub, NEXP), axis=0, keepdims=True)
        top_v.append(mv)
        top_i.append(mi)
        vals = jnp.where(sub == mi, -jnp.inf, vals)

    tv = jnp.concatenate(top_v, axis=0)
    ti = jnp.concatenate(top_i, axis=0)
    ev = jnp.exp(tv - tv[0:1])
    ev = ev / jnp.sum(ev, axis=0, keepdims=True)
    probs_ref[...] = ev
    idx_ref[...] = ti


def _outer(x_hbm, w_vmem, probs_hbm, idx_hbm):
    N, D = x_hbm.shape
    pipeline = pltpu.emit_pipeline(
        functools.partial(_router_block, w_ref=w_vmem),
        grid=(N // R,),
        in_specs=[
            pl.BlockSpec((R, D), lambda i: (i, 0),
                         pipeline_mode=pl.Buffered(buffer_count=NBUF,
                                                   use_lookahead=True)),
        ],
        out_specs=[
            pl.BlockSpec((TOPK, R), lambda i: (0, i)),
            pl.BlockSpec((TOPK, R), lambda i: (0, i)),
        ],
    )
    pipeline(x_hbm, probs_hbm, idx_hbm)


@functools.partial(jax.jit, static_argnames=())
def kernel(x, W):
    B, T, D = x.shape
    N = B * T
    x_flat = x.reshape(N, D)
    probs, idx = pl.pallas_call(
        _outer,
        in_specs=[
            pl.BlockSpec(memory_space=pl.ANY),
            pl.BlockSpec(memory_space=pltpu.VMEM),
        ],
        out_specs=[
            pl.BlockSpec(memory_space=pl.ANY),
            pl.BlockSpec(memory_space=pl.ANY),
        ],
        out_shape=[
            jax.ShapeDtypeStruct((TOPK, N), jnp.float32),
            jax.ShapeDtypeStruct((TOPK, N), jnp.int32),
        ],
    )(x_flat, W)
    aux_loss = jnp.array(0.0, dtype=jnp.float32)
    return (probs.T, idx.T, aux_loss)


# speedup vs baseline: 1.1039x; 1.1039x over previous
import functools

import jax
import jax.numpy as jnp
from jax.experimental import pallas as pl
from jax.experimental.pallas import tpu as pltpu

TOPK = 8
NEXP = 64
R = 512
NBUF = 3


def _router_block(x_ref, probs_ref, idx_ref, w_ref):
    xb = x_ref[...]
    wb = w_ref[...]
    logits = jax.lax.dot_general(
        wb, xb, (((1,), (1,)), ((), ())), preferred_element_type=jnp.float32
    )

    sub = jax.lax.broadcasted_iota(jnp.int32, logits.shape, 0)
    vals = logits
    top_v = []
    top_i = []
    for _ in range(TOPK):
        mv = jnp.max(vals, axis=0, keepdims=True)
        mi = jnp.min(jnp.where(vals == mv, sub, NEXP), axis=0, keepdims=True)
        top_v.append(mv)
        top_i.append(mi)
        vals = jnp.where(sub == mi, -jnp.inf, vals)

    tv = jnp.concatenate(top_v, axis=0)
    ti = jnp.concatenate(top_i, axis=0)
    ev = jnp.exp(tv - tv[0:1])
    ev = ev / jnp.sum(ev, axis=0, keepdims=True)
    probs_ref[...] = ev
    idx_ref[...] = ti


def _outer(x_hbm, w_vmem, probs_hbm, idx_hbm):
    N, D = x_hbm.shape
    pipeline = pltpu.emit_pipeline(
        functools.partial(_router_block, w_ref=w_vmem),
        grid=(N // R,),
        in_specs=[
            pl.BlockSpec((R, D), lambda i: (i, 0),
                         pipeline_mode=pl.Buffered(buffer_count=NBUF,
                                                   use_lookahead=False)),
        ],
        out_specs=[
            pl.BlockSpec((TOPK, R), lambda i: (0, i)),
            pl.BlockSpec((TOPK, R), lambda i: (0, i)),
        ],
    )
    pipeline(x_hbm, probs_hbm, idx_hbm)


@functools.partial(jax.jit, static_argnames=())
def kernel(x, W):
    B, T, D = x.shape
    N = B * T
    x_flat = x.reshape(N, D)
    probs, idx = pl.pallas_call(
        _outer,
        in_specs=[
            pl.BlockSpec(memory_space=pl.ANY),
            pl.BlockSpec(memory_space=pltpu.VMEM),
        ],
        out_specs=[
            pl.BlockSpec(memory_space=pl.ANY),
            pl.BlockSpec(memory_space=pl.ANY),
        ],
        out_shape=[
            jax.ShapeDtypeStruct((TOPK, N), jnp.float32),
            jax.ShapeDtypeStruct((TOPK, N), jnp.int32),
        ],
    )(x_flat, W)
    aux_loss = jnp.array(0.0, dtype=jnp.float32)
    return (probs.T, idx.T, aux_loss)
